# baseline (speedup 1.0000x reference)
import jax
import jax.numpy as jnp
from jax import lax
from jax.experimental import pallas as pl
from jax.experimental.pallas import tpu as pltpu
from jax.experimental.pallas import tpu_sc as plsc

N = 10000
E = 320000
H = 128
HH = H // 2
G = 64
C = 10

NC = 2
NS = 16
K = 128
NCHT = 160
EPT = NCHT * K
EP = NS * EPT
NP = 10240
RPT = NP // NS
CW = 16
NCHW = NCHT // NC


def _sc_layer_body(xs_hbm, idx_hbm, z64_hbm, out_hbm, *scr):
    idx = scr[0:8]
    rows = scr[8:12]
    x_sp, acc = scr[12:14]
    sem_g = scr[14:18]
    sem_i = scr[18:26]
    sem_s = scr[26:30]

    c = lax.axis_index("c")
    s = lax.axis_index("s")

    pltpu.sync_copy(xs_hbm.at[c, pl.ds(s * RPT, RPT)],
                    x_sp.at[pl.ds(s * RPT, RPT)])
    pltpu.sync_copy(z64_hbm, acc.at[pl.ds(s * RPT, RPT)])
    for i in range(4):
        pltpu.sync_copy(idx_hbm.at[s, i], idx[i])
    for i in range(4, 8):
        pltpu.async_copy(idx_hbm.at[s, i], idx[i], sem_i[i])
    plsc.subcore_barrier()

    KH = K // 2
    for i in range(4):
        pltpu.async_copy(x_sp.at[idx[i].at[0, pl.ds(0, KH)]],
                         rows[i].at[pl.ds(0, KH)], sem_g[i])
        pltpu.async_copy(x_sp.at[idx[i].at[0, pl.ds(KH, KH)]],
                         rows[i].at[pl.ds(KH, KH)], sem_g[i])

    def octet(q, carry):
        for i in range(8):
            r = i % 4
            r2 = (i + 2) % 4
            k2 = (i + 2) % 8
            k6 = (i + 6) % 8
            j = 8 * q + i
            pltpu.make_async_copy(xs_hbm.at[0, pl.ds(0, KH)],
                                  rows[r].at[pl.ds(0, KH)], sem_g[r]).wait()
            pltpu.make_async_copy(xs_hbm.at[0, pl.ds(0, KH)],
                                  rows[r].at[pl.ds(KH, KH)], sem_g[r]).wait()
            pltpu.async_copy(rows[r], acc.at[idx[i].at[1]], sem_s[r],
                             add=True)

            @pl.when(jnp.logical_and(j >= 2, j + 2 < NCHT))
            def _():
                pltpu.make_async_copy(rows[r2], acc.at[idx[0].at[1]],
                                      sem_s[r2]).wait()
                pltpu.make_async_copy(idx_hbm.at[0, 0], idx[k2],
                                      sem_i[k2]).wait()
                pltpu.async_copy(x_sp.at[idx[k2].at[0, pl.ds(0, KH)]],
                                 rows[r2].at[pl.ds(0, KH)], sem_g[r2])
                pltpu.async_copy(x_sp.at[idx[k2].at[0, pl.ds(KH, KH)]],
                                 rows[r2].at[pl.ds(KH, KH)], sem_g[r2])

                @pl.when(j + 6 < NCHT)
                def _():
                    pltpu.async_copy(idx_hbm.at[s, j + 6], idx[k6],
                                     sem_i[k6])

        return carry

    lax.fori_loop(0, NCHT // 8, octet, 0)
    for r in range(4):
        pltpu.make_async_copy(rows[r], acc.at[idx[0].at[1]],
                              sem_s[r]).wait()
    plsc.subcore_barrier()
    pltpu.sync_copy(acc.at[pl.ds(s * RPT, RPT)],
                    out_hbm.at[c, pl.ds(s * RPT, RPT)])


_sc_layer = pl.kernel(
    _sc_layer_body,
    out_type=[jax.ShapeDtypeStruct((NC, NP, HH), jnp.float32)],
    mesh=plsc.VectorSubcoreMesh(core_axis_name="c", subcore_axis_name="s"),
    scratch_types=(
        [pltpu.VMEM((2, K), jnp.int32) for _ in range(8)]
        + [pltpu.VMEM((K, HH), jnp.float32) for _ in range(4)]
        + [pltpu.VMEM_SHARED((NP, HH), jnp.float32),
           pltpu.VMEM_SHARED((NP, HH), jnp.float32)]
        + [pltpu.SemaphoreType.DMA] * 16
    ),
    compiler_params=pltpu.CompilerParams(use_tc_tiling_on_sc=False),
)


def _sc_counts_body(idx_hbm, z16_hbm, ones_hbm, cnt_out, idx_v, ones_v, cacc):
    c = lax.axis_index("c")
    s = lax.axis_index("s")

    pltpu.sync_copy(idx_hbm.at[s, pl.ds(c * NCHW, NCHW)], idx_v)
    pltpu.sync_copy(ones_hbm, ones_v)
    pltpu.sync_copy(z16_hbm, cacc.at[pl.ds(s * RPT, RPT)])
    plsc.subcore_barrier()

    def chunk(j, carry):
        pltpu.sync_copy(ones_v, cacc.at[idx_v.at[j, 1]], add=True)
        return carry

    lax.fori_loop(0, NCHW, chunk, 0)
    plsc.subcore_barrier()
    pltpu.sync_copy(cacc.at[pl.ds(s * RPT, RPT)],
                    cnt_out.at[c, pl.ds(s * RPT, RPT)])


_sc_counts = pl.kernel(
    _sc_counts_body,
    out_type=[jax.ShapeDtypeStruct((NC, NP, CW), jnp.float32)],
    mesh=plsc.VectorSubcoreMesh(core_axis_name="c", subcore_axis_name="s"),
    scratch_types=[
        pltpu.VMEM((NCHW, 2, K), jnp.int32),
        pltpu.VMEM((K, CW), jnp.float32),
        pltpu.VMEM_SHARED((NP, CW), jnp.float32),
    ],
    compiler_params=pltpu.CompilerParams(use_tc_tiling_on_sc=False),
)


def _conv1_body(parts, cparts, xs, wn, bn, ws, h_out, csum_out):
    agg = jnp.concatenate([parts[0], parts[1]], axis=1)
    x = jnp.concatenate([xs[0], xs[1]], axis=1)
    cnt = cparts[0] + cparts[1]
    agg = agg / jnp.maximum(cnt[:, 0:1], 1.0)
    hn = lax.dot_general(agg, wn[...], (((1,), (1,)), ((), ())),
                         preferred_element_type=jnp.float32)
    hs = lax.dot_general(x, ws[...], (((1,), (1,)), ((), ())),
                         preferred_element_type=jnp.float32)
    h = jnp.maximum(hn + hs + bn[...], 0.0)
    h_out[...] = jnp.stack([h[:, :HH], h[:, HH:]], axis=0)
    csum_out[...] = cnt


def _conv_body(parts, csum, xs, wn, bn, ws, h_out):
    agg = jnp.concatenate([parts[0], parts[1]], axis=1)
    x = jnp.concatenate([xs[0], xs[1]], axis=1)
    agg = agg / jnp.maximum(csum[:, 0:1], 1.0)
    hn = lax.dot_general(agg, wn[...], (((1,), (1,)), ((), ())),
                         preferred_element_type=jnp.float32)
    hs = lax.dot_general(x, ws[...], (((1,), (1,)), ((), ())),
                         preferred_element_type=jnp.float32)
    h = jnp.maximum(hn + hs + bn[...], 0.0)
    h_out[...] = jnp.stack([h[:, :HH], h[:, HH:]], axis=0)


_B = 1280


def _tc_conv1(parts, cparts, xs, wn, bn, ws):
    grid = (NP // _B,)
    return pl.pallas_call(
        _conv1_body,
        grid=grid,
        in_specs=[
            pl.BlockSpec((NC, _B, HH), lambda i: (0, i, 0)),
            pl.BlockSpec((NC, _B, CW), lambda i: (0, i, 0)),
            pl.BlockSpec((NC, _B, HH), lambda i: (0, i, 0)),
            pl.BlockSpec((H, H), lambda i: (0, 0)),
            pl.BlockSpec((1, H), lambda i: (0, 0)),
            pl.BlockSpec((H, H), lambda i: (0, 0)),
        ],
        out_specs=[
            pl.BlockSpec((NC, _B, HH), lambda i: (0, i, 0)),
            pl.BlockSpec((_B, CW), lambda i: (i, 0)),
        ],
        out_shape=[
            jax.ShapeDtypeStruct((NC, NP, HH), jnp.float32),
            jax.ShapeDtypeStruct((NP, CW), jnp.float32),
        ],
    )(parts, cparts, xs, wn, bn, ws)


def _tc_conv(parts, csum, xs, wn, bn, ws):
    grid = (NP // _B,)
    return pl.pallas_call(
        _conv_body,
        grid=grid,
        in_specs=[
            pl.BlockSpec((NC, _B, HH), lambda i: (0, i, 0)),
            pl.BlockSpec((_B, CW), lambda i: (i, 0)),
            pl.BlockSpec((NC, _B, HH), lambda i: (0, i, 0)),
            pl.BlockSpec((H, H), lambda i: (0, 0)),
            pl.BlockSpec((1, H), lambda i: (0, 0)),
            pl.BlockSpec((H, H), lambda i: (0, 0)),
        ],
        out_specs=pl.BlockSpec((NC, _B, HH), lambda i: (0, i, 0)),
        out_shape=jax.ShapeDtypeStruct((NC, NP, HH), jnp.float32),
    )(parts, csum, xs, wn, bn, ws)


def _final_body(parts, csum, xs, wn, bn, ws, batch, wl, bl, out):
    agg = jnp.concatenate([parts[0, :N], parts[1, :N]], axis=1)
    x = jnp.concatenate([xs[0, :N], xs[1, :N]], axis=1)
    agg = agg / jnp.maximum(csum[:N, 0:1], 1.0)
    hn = lax.dot_general(agg, wn[...], (((1,), (1,)), ((), ())),
                         preferred_element_type=jnp.float32)
    hs = lax.dot_general(x, ws[...], (((1,), (1,)), ((), ())),
                         preferred_element_type=jnp.float32)
    h = jnp.maximum(hn + hs + bn[...], 0.0)
    seg = lax.broadcasted_iota(jnp.int32, (G, N), 0)
    onehot = jnp.where(seg == batch[...], 1.0, 0.0).astype(jnp.float32)
    sums = lax.dot_general(onehot, h, (((1,), (0,)), ((), ())),
                           preferred_element_type=jnp.float32)
    gcnt = jnp.sum(onehot, axis=1, keepdims=True)
    pooled = sums / jnp.maximum(gcnt, 1.0)
    logits = lax.dot_general(pooled, wl[...], (((1,), (1,)), ((), ())),
                             preferred_element_type=jnp.float32) + bl[...]
    shifted = logits - jnp.max(logits, axis=1, keepdims=True)
    out[...] = shifted - jnp.log(jnp.sum(jnp.exp(shifted), axis=1,
                                         keepdims=True))


def _tc_final(parts, csum, xs, wn, bn, ws, batch2d, wl, bl):
    return pl.pallas_call(
        _final_body,
        out_shape=jax.ShapeDtypeStruct((G, C), jnp.float32),
    )(parts, csum, xs, wn, bn, ws, batch2d, wl, bl)


def kernel(x, edge_index, batch, Wn1, bn1, Ws1, Wn2, bn2, Ws2,
           Wn3, bn3, Ws3, Wl, bl):
    pad = EP - E
    src = edge_index[0].astype(jnp.int32)
    dst = edge_index[1].astype(jnp.int32)
    src = jnp.pad(src, (0, pad))
    dst = jnp.concatenate(
        [dst, N + jnp.arange(pad, dtype=jnp.int32) % (NP - N)])
    idx_all = jnp.stack([src.reshape(NS, NCHT, K),
                         dst.reshape(NS, NCHT, K)], axis=2)
    xs = jnp.pad(
        jnp.stack([x[:, :HH], x[:, HH:]], axis=0),
        ((0, 0), (0, NP - N), (0, 0)))
    z64 = jnp.zeros((RPT, HH), jnp.float32)
    z16 = jnp.zeros((RPT, CW), jnp.float32)
    ones16 = jnp.ones((K, CW), jnp.float32)
    bn1r, bn2r, bn3r = (b.reshape(1, H) for b in (bn1, bn2, bn3))
    blr = bl.reshape(1, C)
    batch2d = batch.astype(jnp.int32).reshape(1, N)

    (cparts,) = _sc_counts(idx_all, z16, ones16)
    (parts1,) = _sc_layer(xs, idx_all, z64)
    h1, csum = _tc_conv1(parts1, cparts, xs, Wn1, bn1r, Ws1)
    (parts2,) = _sc_layer(h1, idx_all, z64)
    h2 = _tc_conv(parts2, csum, h1, Wn2, bn2r, Ws2)
    (parts3,) = _sc_layer(h2, idx_all, z64)
    return _tc_final(parts3, csum, h2, Wn3, bn3r, Ws3, batch2d, Wl, blr)


# speedup vs baseline: 1.0030x; 1.0030x over previous
import jax
import jax.numpy as jnp
from jax import lax
from jax.experimental import pallas as pl
from jax.experimental.pallas import tpu as pltpu
from jax.experimental.pallas import tpu_sc as plsc

N = 10000
E = 320000
H = 128
HH = H // 2
G = 64
C = 10

NC = 2
NS = 16
K = 128
NCHT = 160
EPT = NCHT * K
EP = NS * EPT
NP = 10240
RPT = NP // NS
CW = 16
NCHW = NCHT // NC


def _sc_layer_body(xs_hbm, idx_hbm, z64_hbm, out_hbm, *scr):
    idx = scr[0:8]
    rows = scr[8:12]
    x_sp, acc = scr[12:14]
    sem_g = scr[14:18]
    sem_i = scr[18:26]
    sem_s = scr[26:30]

    c = lax.axis_index("c")
    s = lax.axis_index("s")

    pltpu.sync_copy(xs_hbm.at[c, pl.ds(s * RPT, RPT)],
                    x_sp.at[pl.ds(s * RPT, RPT)])
    pltpu.sync_copy(z64_hbm, acc.at[pl.ds(s * RPT, RPT)])
    for i in range(4):
        pltpu.sync_copy(idx_hbm.at[s, i], idx[i])
    for i in range(4, 8):
        pltpu.async_copy(idx_hbm.at[s, i], idx[i], sem_i[i])
    plsc.subcore_barrier()

    KH = K // 2
    for i in range(4):
        pltpu.async_copy(x_sp.at[idx[i].at[0, pl.ds(0, KH)]],
                         rows[i].at[pl.ds(0, KH)], sem_g[i])
        pltpu.async_copy(x_sp.at[idx[i].at[0, pl.ds(KH, KH)]],
                         rows[i].at[pl.ds(KH, KH)], sem_g[i])

    def octet(q, carry):
        for i in range(8):
            r = i % 4
            r2 = (i + 2) % 4
            k2 = (i + 2) % 8
            k6 = (i + 6) % 8
            j = 8 * q + i
            pltpu.make_async_copy(xs_hbm.at[0, pl.ds(0, KH)],
                                  rows[r].at[pl.ds(0, KH)], sem_g[r]).wait()
            pltpu.make_async_copy(xs_hbm.at[0, pl.ds(0, KH)],
                                  rows[r].at[pl.ds(KH, KH)], sem_g[r]).wait()
            pltpu.async_copy(rows[r], acc.at[idx[i].at[1]], sem_s[r],
                             add=True)

            @pl.when(jnp.logical_and(j >= 2, j + 2 < NCHT))
            def _():
                pltpu.make_async_copy(rows[r2], acc.at[idx[0].at[1]],
                                      sem_s[r2]).wait()
                pltpu.make_async_copy(idx_hbm.at[0, 0], idx[k2],
                                      sem_i[k2]).wait()
                pltpu.async_copy(x_sp.at[idx[k2].at[0, pl.ds(0, KH)]],
                                 rows[r2].at[pl.ds(0, KH)], sem_g[r2])
                pltpu.async_copy(x_sp.at[idx[k2].at[0, pl.ds(KH, KH)]],
                                 rows[r2].at[pl.ds(KH, KH)], sem_g[r2])

                @pl.when(j + 6 < NCHT)
                def _():
                    pltpu.async_copy(idx_hbm.at[s, j + 6], idx[k6],
                                     sem_i[k6])

        return carry

    lax.fori_loop(0, NCHT // 8, octet, 0)
    for r in range(4):
        pltpu.make_async_copy(rows[r], acc.at[idx[0].at[1]],
                              sem_s[r]).wait()
    plsc.subcore_barrier()
    pltpu.sync_copy(acc.at[pl.ds(s * RPT, RPT)],
                    out_hbm.at[c, pl.ds(s * RPT, RPT)])


_sc_layer = pl.kernel(
    _sc_layer_body,
    out_type=[jax.ShapeDtypeStruct((NC, NP, HH), jnp.float32)],
    mesh=plsc.VectorSubcoreMesh(core_axis_name="c", subcore_axis_name="s"),
    scratch_types=(
        [pltpu.VMEM((2, K), jnp.int32) for _ in range(8)]
        + [pltpu.VMEM((K, HH), jnp.float32) for _ in range(4)]
        + [pltpu.VMEM_SHARED((NP, HH), jnp.float32),
           pltpu.VMEM_SHARED((NP, HH), jnp.float32)]
        + [pltpu.SemaphoreType.DMA] * 16
    ),
    compiler_params=pltpu.CompilerParams(use_tc_tiling_on_sc=False),
)


def _sc_counts_body(idx_hbm, z16_hbm, ones_hbm, cnt_out,
                    idx_v, ones_v, cacc, sem_c):
    c = lax.axis_index("c")
    s = lax.axis_index("s")

    pltpu.sync_copy(idx_hbm.at[s, pl.ds(c * NCHW, NCHW)], idx_v)
    pltpu.sync_copy(ones_hbm, ones_v)
    pltpu.sync_copy(z16_hbm, cacc.at[pl.ds(s * RPT, RPT)])
    plsc.subcore_barrier()

    def chunk(j, carry):
        @pl.when(j >= 8)
        def _():
            pltpu.make_async_copy(ones_v, cacc.at[idx_v.at[0, 1]],
                                  sem_c).wait()

        pltpu.async_copy(ones_v, cacc.at[idx_v.at[j, 1]], sem_c, add=True)
        return carry

    lax.fori_loop(0, NCHW, chunk, 0)

    def drain(j, carry):
        pltpu.make_async_copy(ones_v, cacc.at[idx_v.at[0, 1]],
                              sem_c).wait()
        return carry

    lax.fori_loop(0, 8, drain, 0)
    plsc.subcore_barrier()
    pltpu.sync_copy(cacc.at[pl.ds(s * RPT, RPT)],
                    cnt_out.at[c, pl.ds(s * RPT, RPT)])


_sc_counts = pl.kernel(
    _sc_counts_body,
    out_type=[jax.ShapeDtypeStruct((NC, NP, CW), jnp.float32)],
    mesh=plsc.VectorSubcoreMesh(core_axis_name="c", subcore_axis_name="s"),
    scratch_types=[
        pltpu.VMEM((NCHW, 2, K), jnp.int32),
        pltpu.VMEM((K, CW), jnp.float32),
        pltpu.VMEM_SHARED((NP, CW), jnp.float32),
        pltpu.SemaphoreType.DMA,
    ],
    compiler_params=pltpu.CompilerParams(use_tc_tiling_on_sc=False),
)


def _conv1_body(parts, cparts, xs, wn, bn, ws, h_out, csum_out):
    agg = jnp.concatenate([parts[0], parts[1]], axis=1)
    x = jnp.concatenate([xs[0], xs[1]], axis=1)
    cnt = cparts[0] + cparts[1]
    agg = agg / jnp.maximum(cnt[:, 0:1], 1.0)
    hn = lax.dot_general(agg, wn[...], (((1,), (1,)), ((), ())),
                         preferred_element_type=jnp.float32)
    hs = lax.dot_general(x, ws[...], (((1,), (1,)), ((), ())),
                         preferred_element_type=jnp.float32)
    h = jnp.maximum(hn + hs + bn[...], 0.0)
    h_out[...] = jnp.stack([h[:, :HH], h[:, HH:]], axis=0)
    csum_out[...] = cnt


def _conv_body(parts, csum, xs, wn, bn, ws, h_out):
    agg = jnp.concatenate([parts[0], parts[1]], axis=1)
    x = jnp.concatenate([xs[0], xs[1]], axis=1)
    agg = agg / jnp.maximum(csum[:, 0:1], 1.0)
    hn = lax.dot_general(agg, wn[...], (((1,), (1,)), ((), ())),
                         preferred_element_type=jnp.float32)
    hs = lax.dot_general(x, ws[...], (((1,), (1,)), ((), ())),
                         preferred_element_type=jnp.float32)
    h = jnp.maximum(hn + hs + bn[...], 0.0)
    h_out[...] = jnp.stack([h[:, :HH], h[:, HH:]], axis=0)


_B = 1280


def _tc_conv1(parts, cparts, xs, wn, bn, ws):
    grid = (NP // _B,)
    return pl.pallas_call(
        _conv1_body,
        grid=grid,
        in_specs=[
            pl.BlockSpec((NC, _B, HH), lambda i: (0, i, 0)),
            pl.BlockSpec((NC, _B, CW), lambda i: (0, i, 0)),
            pl.BlockSpec((NC, _B, HH), lambda i: (0, i, 0)),
            pl.BlockSpec((H, H), lambda i: (0, 0)),
            pl.BlockSpec((1, H), lambda i: (0, 0)),
            pl.BlockSpec((H, H), lambda i: (0, 0)),
        ],
        out_specs=[
            pl.BlockSpec((NC, _B, HH), lambda i: (0, i, 0)),
            pl.BlockSpec((_B, CW), lambda i: (i, 0)),
        ],
        out_shape=[
            jax.ShapeDtypeStruct((NC, NP, HH), jnp.float32),
            jax.ShapeDtypeStruct((NP, CW), jnp.float32),
        ],
    )(parts, cparts, xs, wn, bn, ws)


def _tc_conv(parts, csum, xs, wn, bn, ws):
    grid = (NP // _B,)
    return pl.pallas_call(
        _conv_body,
        grid=grid,
        in_specs=[
            pl.BlockSpec((NC, _B, HH), lambda i: (0, i, 0)),
            pl.BlockSpec((_B, CW), lambda i: (i, 0)),
            pl.BlockSpec((NC, _B, HH), lambda i: (0, i, 0)),
            pl.BlockSpec((H, H), lambda i: (0, 0)),
            pl.BlockSpec((1, H), lambda i: (0, 0)),
            pl.BlockSpec((H, H), lambda i: (0, 0)),
        ],
        out_specs=pl.BlockSpec((NC, _B, HH), lambda i: (0, i, 0)),
        out_shape=jax.ShapeDtypeStruct((NC, NP, HH), jnp.float32),
    )(parts, csum, xs, wn, bn, ws)


def _final_body(parts, csum, xs, wn, bn, ws, batch, wl, bl, out):
    agg = jnp.concatenate([parts[0, :N], parts[1, :N]], axis=1)
    x = jnp.concatenate([xs[0, :N], xs[1, :N]], axis=1)
    agg = agg / jnp.maximum(csum[:N, 0:1], 1.0)
    hn = lax.dot_general(agg, wn[...], (((1,), (1,)), ((), ())),
                         preferred_element_type=jnp.float32)
    hs = lax.dot_general(x, ws[...], (((1,), (1,)), ((), ())),
                         preferred_element_type=jnp.float32)
    h = jnp.maximum(hn + hs + bn[...], 0.0)
    seg = lax.broadcasted_iota(jnp.int32, (G, N), 0)
    onehot = jnp.where(seg == batch[...], 1.0, 0.0).astype(jnp.float32)
    sums = lax.dot_general(onehot, h, (((1,), (0,)), ((), ())),
                           preferred_element_type=jnp.float32)
    gcnt = jnp.sum(onehot, axis=1, keepdims=True)
    pooled = sums / jnp.maximum(gcnt, 1.0)
    logits = lax.dot_general(pooled, wl[...], (((1,), (1,)), ((), ())),
                             preferred_element_type=jnp.float32) + bl[...]
    shifted = logits - jnp.max(logits, axis=1, keepdims=True)
    out[...] = shifted - jnp.log(jnp.sum(jnp.exp(shifted), axis=1,
                                         keepdims=True))


def _tc_final(parts, csum, xs, wn, bn, ws, batch2d, wl, bl):
    return pl.pallas_call(
        _final_body,
        out_shape=jax.ShapeDtypeStruct((G, C), jnp.float32),
    )(parts, csum, xs, wn, bn, ws, batch2d, wl, bl)


def kernel(x, edge_index, batch, Wn1, bn1, Ws1, Wn2, bn2, Ws2,
           Wn3, bn3, Ws3, Wl, bl):
    pad = EP - E
    src = edge_index[0].astype(jnp.int32)
    dst = edge_index[1].astype(jnp.int32)
    src = jnp.pad(src, (0, pad))
    dst = jnp.concatenate(
        [dst, N + jnp.arange(pad, dtype=jnp.int32) % (NP - N)])
    idx_all = jnp.stack([src.reshape(NS, NCHT, K),
                         dst.reshape(NS, NCHT, K)], axis=2)
    xs = jnp.pad(
        jnp.stack([x[:, :HH], x[:, HH:]], axis=0),
        ((0, 0), (0, NP - N), (0, 0)))
    z64 = jnp.zeros((RPT, HH), jnp.float32)
    z16 = jnp.zeros((RPT, CW), jnp.float32)
    ones16 = jnp.ones((K, CW), jnp.float32)
    bn1r, bn2r, bn3r = (b.reshape(1, H) for b in (bn1, bn2, bn3))
    blr = bl.reshape(1, C)
    batch2d = batch.astype(jnp.int32).reshape(1, N)

    (cparts,) = _sc_counts(idx_all, z16, ones16)
    (parts1,) = _sc_layer(xs, idx_all, z64)
    h1, csum = _tc_conv1(parts1, cparts, xs, Wn1, bn1r, Ws1)
    (parts2,) = _sc_layer(h1, idx_all, z64)
    h2 = _tc_conv(parts2, csum, h1, Wn2, bn2r, Ws2)
    (parts3,) = _sc_layer(h2, idx_all, z64)
    return _tc_final(parts3, csum, h2, Wn3, bn3r, Ws3, batch2d, Wl, blr)
